# baseline (speedup 1.0000x reference)
import functools
import math

import jax
import jax.numpy as jnp
from jax import lax
from jax.experimental import pallas as pl
from jax.experimental.pallas import tpu as pltpu
from jax.experimental.pallas import tpu_sc as plsc

_PAD = 0
_SMOOTHING = 0.1
_CONFIDENCE = 1.0 - _SMOOTHING

_NC = 2
_NS = 16
_NW = _NC * _NS
_L = 16

_N_SC_ROWS = 1280
_SC_CHUNK = 4096


def _make_sc_kernel(n, v):
    bpw = n // _NW
    batch = 32
    rpt = _N_SC_ROWS // _NW
    n_tc = n - _N_SC_ROWS
    ngrp = rpt // 8
    nch = v // _SC_CHUNK
    mesh = plsc.VectorSubcoreMesh(core_axis_name="c", subcore_axis_name="s")

    @functools.partial(
        pl.kernel,
        mesh=mesh,
        out_type=jax.ShapeDtypeStruct((_NW, 3 * _L), jnp.float32),
        scratch_types=[
            pltpu.VMEM((bpw,), jnp.int32),
            pltpu.VMEM((rpt + 8,), jnp.int32),
            pltpu.VMEM((batch, 8, 128), jnp.float32),
            pltpu.VMEM((2, 8, _SC_CHUNK), jnp.float32),
            pltpu.VMEM((3 * _L,), jnp.float32),
            pltpu.SemaphoreType.DMA,
            pltpu.SemaphoreType.DMA,
        ],
    )
    def sc_kernel(x_hbm, tgt_hbm, out_hbm, tgtv, tgt3v, bufs, ring, st,
                  sem0, sem1):
        wid = lax.axis_index("s") * _NC + lax.axis_index("c")
        base = wid * bpw
        pltpu.sync_copy(tgt_hbm.at[pl.ds(base, bpw)], tgtv)
        pltpu.sync_copy(tgt_hbm.at[pl.ds(n_tc + wid * rpt, rpt)],
                        tgt3v.at[pl.ds(0, rpt)])
        lanes = lax.broadcasted_iota(jnp.int32, (_L,), 0)
        onehot0 = jnp.where(lanes == 0, 1.0, 0.0)

        accg = jnp.zeros((_L,), jnp.float32)
        accn = jnp.zeros((_L,), jnp.float32)
        for b0 in range(0, bpw, batch):
            copies = []
            for c in range(batch // _L):
                tv = tgtv[pl.ds(b0 + c * _L, _L)]
                for q in range(_L):
                    k = b0 + c * _L + q
                    t = tv[q]
                    col0 = pl.multiple_of(jnp.bitwise_and(t, -128), 128)
                    cp = pltpu.make_async_copy(
                        x_hbm.at[pl.ds(base + (k & ~7), 8), pl.ds(col0, 128)],
                        bufs.at[k - b0], sem0)
                    cp.start()
                    copies.append(cp)
            for cp in copies:
                cp.wait()
            for c in range(batch // _L):
                tv = tgtv[pl.ds(b0 + c * _L, _L)]
                for q in range(_L):
                    k = b0 + c * _L + q
                    t = tv[q]
                    w0 = jnp.bitwise_and(t, 127 - (_L - 1))
                    vals = bufs[k - b0, k & 7, pl.ds(w0, _L)]
                    sel = lanes == jnp.bitwise_and(t, _L - 1)
                    validf = (t != _PAD).astype(jnp.float32)
                    accg = accg + jnp.where(sel, vals, 0.0) * validf
                    accn = accn + onehot0 * validf


        acc3 = jnp.zeros((_L,), jnp.float32)
        for g in range(ngrp):
            r0 = pl.multiple_of(n_tc + wid * rpt + g * 8, 8)

            def issue(c):
                return pltpu.make_async_copy(
                    x_hbm.at[pl.ds(r0, 8), pl.ds(c * _SC_CHUNK, _SC_CHUNK)],
                    ring.at[c % 2], sem0 if c % 2 == 0 else sem1)

            issue(0).start()
            raccs = [jnp.zeros((_L,), jnp.float32) for _ in range(8)]
            x0vec = [None] * 8
            for c in range(nch):
                slot = c % 2
                if c + 1 < nch:
                    issue(c + 1).start()
                issue(c).wait()

                def body(j, accs, _slot=slot):
                    off = j * 32
                    return tuple(
                        accs[r]
                        + ring[_slot, r, pl.ds(off, _L)]
                        + ring[_slot, r, pl.ds(off + _L, _L)]
                        for r in range(8))

                raccs = list(lax.fori_loop(0, _SC_CHUNK // 32, body,
                                           tuple(raccs)))
                if c == 0:
                    for r in range(8):
                        x0vec[r] = onehot0 * ring[0, r, pl.ds(0, _L)]
            tv16 = tgt3v[pl.ds((g // 2) * _L, _L)]
            for r in range(8):
                t = tv16[(g % 2) * 8 + r]
                validf = (t != _PAD).astype(jnp.float32)
                acc3 = acc3 + (raccs[r] - x0vec[r]) * validf
        st[pl.ds(0, _L)] = accg
        st[pl.ds(_L, _L)] = accn
        st[pl.ds(2 * _L, _L)] = acc3
        pltpu.sync_copy(st, out_hbm.at[wid])

    return sc_kernel


def _tc_body(tgt_ref, x_ref, out_ref, acc_ref, *, nrb, smooth):
    j = pl.program_id(0)
    t = tgt_ref[...]
    validf = (t != _PAD).astype(jnp.float32)
    xb = x_ref[...]

    rs = jnp.sum(xb, axis=1, keepdims=True)
    contrib = jnp.sum(validf * (rs - xb[:, 0:1]))

    @pl.when(j == 0)
    def _init():
        acc_ref[0] = contrib

    @pl.when(j > 0)
    def _accum():
        acc_ref[0] = acc_ref[0] + contrib

    @pl.when(j == nrb - 1)
    def _finish():
        out_ref[0, 0] = acc_ref[0]


def kernel(x, target):
    x2 = x.reshape(-1, x.shape[-1])
    n, v = x2.shape
    tgt = target.reshape(-1).astype(jnp.int32)
    smooth = _SMOOTHING / (v - 2)
    n_tc = n - _N_SC_ROWS

    sc_partials = _make_sc_kernel(n, v)(x2, tgt)

    rb = 128
    nrb = n_tc // rb
    acc_tc = pl.pallas_call(
        functools.partial(_tc_body, nrb=nrb, smooth=smooth),
        grid=(nrb,),
        in_specs=[
            pl.BlockSpec((rb, 1), lambda j: (j, 0)),
            pl.BlockSpec((rb, v), lambda j: (j, 0)),
        ],
        out_specs=pl.BlockSpec(memory_space=pltpu.SMEM),
        out_shape=jax.ShapeDtypeStruct((1, 1), jnp.float32),
        scratch_shapes=[
            pltpu.SMEM((1,), jnp.float32),
        ],
    )(tgt.reshape(n, 1)[:n_tc], x2)

    sums = jnp.sum(sc_partials.reshape(_NW, 3, _L), axis=(0, 2))
    g, nv, a_sc = sums[0], sums[1], sums[2]
    k_const = ((v - 2) * smooth * math.log(smooth)
               + _CONFIDENCE * math.log(_CONFIDENCE))
    return (k_const * nv - smooth * (acc_tc[0, 0] + a_sc)
            + (smooth - _CONFIDENCE) * g) / nv


# speedup vs baseline: 1.0386x; 1.0386x over previous
import functools
import math

import jax
import jax.numpy as jnp
from jax import lax
from jax.experimental import pallas as pl
from jax.experimental.pallas import tpu as pltpu
from jax.experimental.pallas import tpu_sc as plsc

_PAD = 0
_SMOOTHING = 0.1
_CONFIDENCE = 1.0 - _SMOOTHING

_NC = 2
_NS = 16
_NW = _NC * _NS
_L = 16


def _make_sc_gather(n, v):
    bpw = n // _NW
    batch = 64
    mesh = plsc.VectorSubcoreMesh(core_axis_name="c", subcore_axis_name="s")

    @functools.partial(
        pl.kernel,
        mesh=mesh,
        out_type=jax.ShapeDtypeStruct((_NW, _L), jnp.float32),
        scratch_types=[
            pltpu.VMEM((bpw,), jnp.int32),
            pltpu.VMEM((batch, 8, 128), jnp.float32),
            pltpu.VMEM((_L,), jnp.float32),
            pltpu.SemaphoreType.DMA,
        ],
    )
    def sc_gather(x_hbm, tgt_hbm, out_hbm, tgtv, bufs, stage, sem):
        wid = lax.axis_index("s") * _NC + lax.axis_index("c")
        base = wid * bpw
        pltpu.sync_copy(tgt_hbm.at[pl.ds(base, bpw)], tgtv)
        lanes = lax.broadcasted_iota(jnp.int32, (_L,), 0)
        acc = jnp.zeros((_L,), jnp.float32)
        for b0 in range(0, bpw, batch):
            copies = []
            for c in range(batch // _L):
                tv = tgtv[pl.ds(b0 + c * _L, _L)]
                for q in range(_L):
                    k = b0 + c * _L + q
                    t = tv[q]
                    col0 = pl.multiple_of(jnp.bitwise_and(t, -128), 128)
                    cp = pltpu.make_async_copy(
                        x_hbm.at[pl.ds(base + (k & ~7), 8), pl.ds(col0, 128)],
                        bufs.at[k - b0], sem)
                    cp.start()
                    copies.append(cp)
            for cp in copies:
                cp.wait()
            for c in range(batch // _L):
                tv = tgtv[pl.ds(b0 + c * _L, _L)]
                for q in range(_L):
                    k = b0 + c * _L + q
                    t = tv[q]
                    w0 = jnp.bitwise_and(t, 127 - (_L - 1))
                    vals = bufs[k - b0, k & 7, pl.ds(w0, _L)]
                    sel = lanes == jnp.bitwise_and(t, _L - 1)
                    validf = (t != _PAD).astype(jnp.float32)
                    acc = acc + jnp.where(sel, vals, 0.0) * validf
        stage[...] = acc
        pltpu.sync_copy(stage, out_hbm.at[wid])

    return sc_gather


def _tc_body(tgt_ref, x_ref, out_ref, nv_ref, acc_ref, nvacc_ref,
             *, nrb, smooth):
    j = pl.program_id(0)
    t = tgt_ref[...]
    validf = (t != _PAD).astype(jnp.float32)
    xb = x_ref[...]

    rs = jnp.sum(xb, axis=1, keepdims=True)
    contrib = jnp.sum(validf * (rs - xb[:, 0:1]))
    nv_part = jnp.sum(validf)

    @pl.when(j == 0)
    def _init():
        acc_ref[0] = contrib
        nvacc_ref[0] = nv_part

    @pl.when(j > 0)
    def _accum():
        acc_ref[0] = acc_ref[0] + contrib
        nvacc_ref[0] = nvacc_ref[0] + nv_part

    @pl.when(j == nrb - 1)
    def _finish():
        v = x_ref.shape[1]
        k_const = ((v - 2) * smooth * math.log(smooth)
                   + _CONFIDENCE * math.log(_CONFIDENCE))
        nv = nvacc_ref[0]
        nv_ref[0, 0] = nv
        out_ref[0, 0] = -smooth * acc_ref[0] + k_const * nv


def kernel(x, target):
    x2 = x.reshape(-1, x.shape[-1])
    n, v = x2.shape
    tgt = target.reshape(-1).astype(jnp.int32)
    smooth = _SMOOTHING / (v - 2)

    sc_partials = _make_sc_gather(n, v)(x2, tgt)

    rb = 128
    nrb = n // rb
    acc, nv = pl.pallas_call(
        functools.partial(_tc_body, nrb=nrb, smooth=smooth),
        grid=(nrb,),
        in_specs=[
            pl.BlockSpec((rb, 1), lambda j: (j, 0)),
            pl.BlockSpec((rb, v), lambda j: (j, 0)),
        ],
        out_specs=[
            pl.BlockSpec(memory_space=pltpu.SMEM),
            pl.BlockSpec(memory_space=pltpu.SMEM),
        ],
        out_shape=[
            jax.ShapeDtypeStruct((1, 1), jnp.float32),
            jax.ShapeDtypeStruct((1, 1), jnp.float32),
        ],
        scratch_shapes=[
            pltpu.SMEM((1,), jnp.float32),
            pltpu.SMEM((1,), jnp.float32),
        ],
    )(tgt.reshape(n, 1), x2)
    g = jnp.sum(sc_partials)
    return (acc[0, 0] + (smooth - _CONFIDENCE) * g) / nv[0, 0]


# speedup vs baseline: 1.0561x; 1.0169x over previous
import functools
import math

import jax
import jax.numpy as jnp
from jax import lax
from jax.experimental import pallas as pl
from jax.experimental.pallas import tpu as pltpu
from jax.experimental.pallas import tpu_sc as plsc

_PAD = 0
_SMOOTHING = 0.1
_CONFIDENCE = 1.0 - _SMOOTHING

_NC = 2
_NS = 16
_NW = _NC * _NS
_L = 16


def _make_sc_gather(n, v):
    bpw = n // _NW
    batch = 64
    mesh = plsc.VectorSubcoreMesh(core_axis_name="c", subcore_axis_name="s")

    @functools.partial(
        pl.kernel,
        mesh=mesh,
        out_type=jax.ShapeDtypeStruct((_NW, _L), jnp.float32),
        scratch_types=[
            pltpu.VMEM((bpw,), jnp.int32),
            pltpu.VMEM((batch, 8, 128), jnp.float32),
            pltpu.VMEM((_L,), jnp.float32),
            pltpu.SemaphoreType.DMA,
        ],
    )
    def sc_gather(x_hbm, tgt_hbm, out_hbm, tgtv, bufs, stage, sem):
        wid = lax.axis_index("s") * _NC + lax.axis_index("c")
        base = wid * bpw
        pltpu.sync_copy(tgt_hbm.at[pl.ds(base, bpw)], tgtv)
        lanes = lax.broadcasted_iota(jnp.int32, (_L,), 0)
        acc = jnp.zeros((_L,), jnp.float32)
        for b0 in range(0, bpw, batch):
            copies = []
            for c in range(batch // _L):
                tv = tgtv[pl.ds(b0 + c * _L, _L)]
                for q in range(_L):
                    k = b0 + c * _L + q
                    t = tv[q]
                    col0 = pl.multiple_of(jnp.bitwise_and(t, -128), 128)
                    cp = pltpu.make_async_copy(
                        x_hbm.at[pl.ds(base + (k & ~7), 8), pl.ds(col0, 128)],
                        bufs.at[k - b0], sem)
                    cp.start()
                    copies.append(cp)
            for cp in copies:
                cp.wait()
            for c in range(batch // _L):
                tv = tgtv[pl.ds(b0 + c * _L, _L)]
                for q in range(_L):
                    k = b0 + c * _L + q
                    t = tv[q]
                    w0 = jnp.bitwise_and(t, 127 - (_L - 1))
                    vals = bufs[k - b0, k & 7, pl.ds(w0, _L)]
                    sel = lanes == jnp.bitwise_and(t, _L - 1)
                    validf = (t != _PAD).astype(jnp.float32)
                    acc = acc + jnp.where(sel, vals, 0.0) * validf
        stage[...] = acc
        pltpu.sync_copy(stage, out_hbm.at[wid])

    return sc_gather


def _tc_body(tgt_ref, x_ref, out_ref, nv_ref, acc_ref, nvacc_ref,
             *, nrb, smooth):
    j = pl.program_id(0)
    t = tgt_ref[...]
    validf = (t != _PAD).astype(jnp.float32)
    xb = x_ref[...]

    rs = jnp.sum(xb, axis=1)
    contrib = jnp.sum(validf * (rs - xb[:, 0]))
    nv_part = jnp.sum(validf)

    @pl.when(j == 0)
    def _init():
        acc_ref[0] = contrib
        nvacc_ref[0] = nv_part

    @pl.when(j > 0)
    def _accum():
        acc_ref[0] = acc_ref[0] + contrib
        nvacc_ref[0] = nvacc_ref[0] + nv_part

    @pl.when(j == nrb - 1)
    def _finish():
        v = x_ref.shape[1]
        k_const = ((v - 2) * smooth * math.log(smooth)
                   + _CONFIDENCE * math.log(_CONFIDENCE))
        nv = nvacc_ref[0]
        nv_ref[0, 0] = nv
        out_ref[0, 0] = -smooth * acc_ref[0] + k_const * nv


def kernel(x, target):
    x2 = x.reshape(-1, x.shape[-1])
    n, v = x2.shape
    tgt = target.reshape(-1).astype(jnp.int32)
    smooth = _SMOOTHING / (v - 2)

    sc_partials = _make_sc_gather(n, v)(x2, tgt)

    rb = 128
    nrb = n // rb
    acc, nv = pl.pallas_call(
        functools.partial(_tc_body, nrb=nrb, smooth=smooth),
        grid=(nrb,),
        in_specs=[
            pl.BlockSpec((rb,), lambda j: (j,)),
            pl.BlockSpec((rb, v), lambda j: (j, 0)),
        ],
        out_specs=[
            pl.BlockSpec(memory_space=pltpu.SMEM),
            pl.BlockSpec(memory_space=pltpu.SMEM),
        ],
        out_shape=[
            jax.ShapeDtypeStruct((1, 1), jnp.float32),
            jax.ShapeDtypeStruct((1, 1), jnp.float32),
        ],
        scratch_shapes=[
            pltpu.SMEM((1,), jnp.float32),
            pltpu.SMEM((1,), jnp.float32),
        ],
    )(tgt, x2)
    g = jnp.sum(sc_partials)
    return (acc[0, 0] + (smooth - _CONFIDENCE) * g) / nv[0, 0]
